# Initial kernel scaffold; baseline (speedup 1.0000x reference)
import jax
import jax.numpy as jnp
from jax.experimental import pallas as pl


def kernel(user_emb, item_emb, adj_val, adj_row, adj_col, users, pos_items, neg_items):
    raise NotImplementedError("write your pallas kernel here")


# speedup vs baseline: 7.2093x; 7.2093x over previous
import functools

import jax
import jax.numpy as jnp
from jax import lax
from jax.experimental import pallas as pl
from jax.experimental.pallas import tpu as pltpu
from jax.experimental.pallas import tpu_sc as plsc

NC = 2
NS = 16
L = 16

N_NODES = 100000
NPAD = 100096
EMB = 32
HALF = EMB // 2
NNZ = 1600000
N_LAYERS = 3
BATCH = 4096

CHUNK = 128
KB = 8
NE_PAD = 1605632
BLOCKS = NE_PAD // (NS * KB * CHUNK)
EROWS = NE_PAD // CHUNK

ROWS_PER_SUB = NPAD // NS
ZCH = 368
ZIT = ROWS_PER_SUB // ZCH

BCH = BATCH // CHUNK
BROWS_PER_SUB = BCH // NS


def _sc_body(e0, col2, row2, val2, users2, pos2, neg2,
             e1, e2, e3, u_out, p_out, n_out,
             acc, col_v, row_v, val_v, cidx_v, fidx_v,
             gbuf_v, g2_v, obuf_v, zbuf_v, sem):
  c = lax.axis_index("c")
  s = lax.axis_index("s")
  c_off = c * NPAD
  tabs = [e0, e1, e2, e3]

  @pl.loop(0, ZCH)
  def _zero_stage(r):
    zbuf_v[r, :] = jnp.zeros((L,), jnp.float32)

  base_s = s * ROWS_PER_SUB

  for l in range(N_LAYERS):
    src = tabs[l]
    dst = tabs[l + 1]

    @pl.loop(0, ZIT)
    def _clear(z):
      pltpu.sync_copy(zbuf_v, acc.at[pl.ds(base_s + z * ZCH, ZCH), :])

    plsc.subcore_barrier()

    tile_row0 = s * (BLOCKS * KB)

    @pl.loop(0, BLOCKS)
    def _edge_block(b):
      rbase = tile_row0 + b * KB
      pltpu.sync_copy(col2.at[pl.ds(rbase, KB), :], col_v)
      pltpu.sync_copy(row2.at[pl.ds(rbase, KB), :], row_v)
      pltpu.sync_copy(val2.at[pl.ds(rbase, KB), :], val_v)
      for j in range(KB):
        for v in range(CHUNK // L):
          cidx_v[pl.ds(v * L, L)] = col_v[j, pl.ds(v * L, L)] + c_off
        pltpu.async_copy(src.at[cidx_v], gbuf_v, sem).wait()

        @pl.loop(0, CHUNK // L)
        def _scale(g):
          vv = val_v[j, pl.ds(g * L, L)]
          for ee in range(L):
            e = g * L + ee
            gbuf_v[e, :] = gbuf_v[e, :] * vv[ee]

        pltpu.sync_copy(gbuf_v, acc.at[row_v.at[j]], add=True)

    plsc.subcore_barrier()

    @pl.loop(0, ZIT)
    def _copy_out(z):
      off = base_s + z * ZCH
      pltpu.sync_copy(acc.at[pl.ds(off, ZCH), :],
                      dst.at[pl.ds(c_off + off, ZCH), :])

    plsc.subcore_barrier()

  for idx_hbm, out_hbm in ((users2, u_out), (pos2, p_out), (neg2, n_out)):
    for rr in range(BROWS_PER_SUB):
      r = s * BROWS_PER_SUB + rr
      pltpu.sync_copy(idx_hbm.at[pl.ds(r * CHUNK, CHUNK)], fidx_v)
      for v in range(CHUNK // L):
        cidx_v[pl.ds(v * L, L)] = fidx_v[pl.ds(v * L, L)] + c_off
      pltpu.async_copy(tabs[0].at[cidx_v], obuf_v, sem).wait()
      for l in range(1, N_LAYERS + 1):
        pltpu.async_copy(tabs[l].at[cidx_v], g2_v, sem).wait()

        @pl.loop(0, CHUNK, unroll=8)
        def _acc_l(e):
          obuf_v[e, :] = obuf_v[e, :] + g2_v[e, :]

      @pl.loop(0, CHUNK, unroll=8)
      def _mean(e):
        obuf_v[e, :] = obuf_v[e, :] * jnp.float32(1.0 / (N_LAYERS + 1))

      pltpu.sync_copy(obuf_v,
                      out_hbm.at[pl.ds(c * BATCH + r * CHUNK, CHUNK), :])


@jax.jit
def _lightgcn(e0, col2, row2, val2, users2, pos2, neg2):
  mesh = plsc.VectorSubcoreMesh(core_axis_name="c", subcore_axis_name="s",
                                num_cores=NC, num_subcores=NS)
  f32 = jnp.float32
  out_type = [
      jax.ShapeDtypeStruct((NC * NPAD, HALF), f32),
      jax.ShapeDtypeStruct((NC * NPAD, HALF), f32),
      jax.ShapeDtypeStruct((NC * NPAD, HALF), f32),
      jax.ShapeDtypeStruct((NC * BATCH, HALF), f32),
      jax.ShapeDtypeStruct((NC * BATCH, HALF), f32),
      jax.ShapeDtypeStruct((NC * BATCH, HALF), f32),
  ]
  scratch = [
      pltpu.VMEM_SHARED((NPAD, HALF), f32),
      pltpu.VMEM((KB, CHUNK), jnp.int32),
      pltpu.VMEM((KB, CHUNK), jnp.int32),
      pltpu.VMEM((KB, CHUNK), f32),
      pltpu.VMEM((CHUNK,), jnp.int32),
      pltpu.VMEM((CHUNK,), jnp.int32),
      pltpu.VMEM((CHUNK, HALF), f32),
      pltpu.VMEM((CHUNK, HALF), f32),
      pltpu.VMEM((CHUNK, HALF), f32),
      pltpu.VMEM((ZCH, HALF), f32),
      pltpu.SemaphoreType.DMA,
  ]
  fn = pl.kernel(_sc_body, out_type=out_type, mesh=mesh,
                 scratch_types=scratch,
                 compiler_params=pltpu.CompilerParams(
                     use_tc_tiling_on_sc=False))
  return fn(e0, col2, row2, val2, users2, pos2, neg2)


def kernel(user_emb, item_emb, adj_val, adj_row, adj_col,
           users, pos_items, neg_items):
  n_user = user_emb.shape[0]
  E = jnp.concatenate([user_emb, item_emb], axis=0)
  zpad = jnp.zeros((NPAD - N_NODES, HALF), jnp.float32)
  e0 = jnp.concatenate([E[:, :HALF], zpad, E[:, HALF:], zpad], axis=0)

  pad = NE_PAD - NNZ
  col2 = jnp.concatenate([adj_col, jnp.zeros((pad,), jnp.int32)])
  row2 = jnp.concatenate([adj_row, jnp.zeros((pad,), jnp.int32)])
  val2 = jnp.concatenate([adj_val, jnp.zeros((pad,), jnp.float32)])
  col2 = col2.reshape(EROWS, CHUNK)
  row2 = row2.reshape(EROWS, CHUNK)
  val2 = val2.reshape(EROWS, CHUNK)

  users2 = users
  pos2 = pos_items + n_user
  neg2 = neg_items + n_user

  _, _, _, u_o, p_o, n_o = _lightgcn(e0, col2, row2, val2,
                                     users2, pos2, neg2)
  u_emb = jnp.concatenate([u_o[:BATCH], u_o[BATCH:]], axis=1)
  pos_emb = jnp.concatenate([p_o[:BATCH], p_o[BATCH:]], axis=1)
  neg_emb = jnp.concatenate([n_o[:BATCH], n_o[BATCH:]], axis=1)
  return (u_emb, pos_emb, neg_emb)


# speedup vs baseline: 14.4643x; 2.0063x over previous
import functools

import jax
import jax.numpy as jnp
from jax import lax
from jax.experimental import pallas as pl
from jax.experimental.pallas import tpu as pltpu
from jax.experimental.pallas import tpu_sc as plsc

NC = 2
NS = 16
L = 16

N_NODES = 100000
NPAD = 100096
EMB = 32
HALF = EMB // 2
NNZ = 1600000
N_LAYERS = 3
BATCH = 4096

CHUNK = 128
KB = 8
NE_PAD = 1605632
BLOCKS = NE_PAD // (NS * KB * CHUNK)
EROWS = NE_PAD // CHUNK

ROWS_PER_SUB = NPAD // NS
ZCH = 184
ZIT = ROWS_PER_SUB // ZCH

BCH = BATCH // CHUNK
BROWS_PER_SUB = BCH // NS


def _sc_body(e0, col2, row2, val2, users2, pos2, neg2,
             e1, e2, e3, u_out, p_out, n_out,
             acc, col_v, row_v, val_v, cidx_v, fidx_v,
             gbuf_v, g2_v, obuf_v, zbuf_v, sem, ssem):
  c = lax.axis_index("c")
  s = lax.axis_index("s")
  c_off = c * NPAD
  tabs = [e0, e1, e2, e3]

  @pl.loop(0, ZCH)
  def _zero_stage(r):
    zbuf_v[r, :] = jnp.zeros((L,), jnp.float32)

  base_s = s * ROWS_PER_SUB

  for l in range(N_LAYERS):
    src = tabs[l]
    dst = tabs[l + 1]

    @pl.loop(0, ZIT)
    def _clear(z):
      pltpu.sync_copy(zbuf_v, acc.at[pl.ds(base_s + z * ZCH, ZCH), :])

    plsc.subcore_barrier()

    tile_row0 = s * (BLOCKS * KB)

    @pl.loop(0, BLOCKS)
    def _edge_block(b):
      rbase = tile_row0 + b * KB
      pltpu.sync_copy(col2.at[pl.ds(rbase, KB), :], col_v)
      pltpu.sync_copy(row2.at[pl.ds(rbase, KB), :], row_v)
      pltpu.sync_copy(val2.at[pl.ds(rbase, KB), :], val_v)
      for j in range(KB):
        for v in range(CHUNK // L):
          cidx_v[j, pl.ds(v * L, L)] = col_v[j, pl.ds(v * L, L)] + c_off
      gds = [pltpu.async_copy(src.at[cidx_v.at[j]], gbuf_v.at[j], sem)
             for j in range(KB)]
      sds = []
      for j in range(KB):
        gds[j].wait()

        @pl.loop(0, CHUNK // L)
        def _scale(g):
          vv = val_v[j, pl.ds(g * L, L)]
          for ee in range(L):
            e = g * L + ee
            gbuf_v[j, e, :] = gbuf_v[j, e, :] * vv[ee]

        sds.append(pltpu.async_copy(gbuf_v.at[j], acc.at[row_v.at[j]],
                                    ssem, add=True))
      for d in sds:
        d.wait()

    plsc.subcore_barrier()

    @pl.loop(0, ZIT)
    def _copy_out(z):
      off = base_s + z * ZCH
      pltpu.sync_copy(acc.at[pl.ds(off, ZCH), :],
                      dst.at[pl.ds(c_off + off, ZCH), :])

    plsc.subcore_barrier()

  for idx_hbm, out_hbm in ((users2, u_out), (pos2, p_out), (neg2, n_out)):
    for rr in range(BROWS_PER_SUB):
      r = s * BROWS_PER_SUB + rr
      pltpu.sync_copy(idx_hbm.at[pl.ds(r * CHUNK, CHUNK)], fidx_v)
      for v in range(CHUNK // L):
        cidx_v[0, pl.ds(v * L, L)] = fidx_v[pl.ds(v * L, L)] + c_off
      pltpu.async_copy(tabs[0].at[cidx_v.at[0]], obuf_v, sem).wait()
      for l in range(1, N_LAYERS + 1):
        pltpu.async_copy(tabs[l].at[cidx_v.at[0]], g2_v, sem).wait()

        @pl.loop(0, CHUNK, unroll=8)
        def _acc_l(e):
          obuf_v[e, :] = obuf_v[e, :] + g2_v[e, :]

      @pl.loop(0, CHUNK, unroll=8)
      def _mean(e):
        obuf_v[e, :] = obuf_v[e, :] * jnp.float32(1.0 / (N_LAYERS + 1))

      pltpu.sync_copy(obuf_v,
                      out_hbm.at[pl.ds(c * BATCH + r * CHUNK, CHUNK), :])


@jax.jit
def _lightgcn(e0, col2, row2, val2, users2, pos2, neg2):
  mesh = plsc.VectorSubcoreMesh(core_axis_name="c", subcore_axis_name="s",
                                num_cores=NC, num_subcores=NS)
  f32 = jnp.float32
  out_type = [
      jax.ShapeDtypeStruct((NC * NPAD, HALF), f32),
      jax.ShapeDtypeStruct((NC * NPAD, HALF), f32),
      jax.ShapeDtypeStruct((NC * NPAD, HALF), f32),
      jax.ShapeDtypeStruct((NC * BATCH, HALF), f32),
      jax.ShapeDtypeStruct((NC * BATCH, HALF), f32),
      jax.ShapeDtypeStruct((NC * BATCH, HALF), f32),
  ]
  scratch = [
      pltpu.VMEM_SHARED((NPAD, HALF), f32),
      pltpu.VMEM((KB, CHUNK), jnp.int32),
      pltpu.VMEM((KB, CHUNK), jnp.int32),
      pltpu.VMEM((KB, CHUNK), f32),
      pltpu.VMEM((KB, CHUNK), jnp.int32),
      pltpu.VMEM((CHUNK,), jnp.int32),
      pltpu.VMEM((KB, CHUNK, HALF), f32),
      pltpu.VMEM((CHUNK, HALF), f32),
      pltpu.VMEM((CHUNK, HALF), f32),
      pltpu.VMEM((ZCH, HALF), f32),
      pltpu.SemaphoreType.DMA,
      pltpu.SemaphoreType.DMA,
  ]
  fn = pl.kernel(_sc_body, out_type=out_type, mesh=mesh,
                 scratch_types=scratch,
                 compiler_params=pltpu.CompilerParams(
                     use_tc_tiling_on_sc=False))
  return fn(e0, col2, row2, val2, users2, pos2, neg2)


def kernel(user_emb, item_emb, adj_val, adj_row, adj_col,
           users, pos_items, neg_items):
  n_user = user_emb.shape[0]
  E = jnp.concatenate([user_emb, item_emb], axis=0)
  zpad = jnp.zeros((NPAD - N_NODES, HALF), jnp.float32)
  e0 = jnp.concatenate([E[:, :HALF], zpad, E[:, HALF:], zpad], axis=0)

  pad = NE_PAD - NNZ
  col2 = jnp.concatenate([adj_col, jnp.zeros((pad,), jnp.int32)])
  row2 = jnp.concatenate([adj_row, jnp.zeros((pad,), jnp.int32)])
  val2 = jnp.concatenate([adj_val, jnp.zeros((pad,), jnp.float32)])
  col2 = col2.reshape(EROWS, CHUNK)
  row2 = row2.reshape(EROWS, CHUNK)
  val2 = val2.reshape(EROWS, CHUNK)

  users2 = users
  pos2 = pos_items + n_user
  neg2 = neg_items + n_user

  _, _, _, u_o, p_o, n_o = _lightgcn(e0, col2, row2, val2,
                                     users2, pos2, neg2)
  u_emb = jnp.concatenate([u_o[:BATCH], u_o[BATCH:]], axis=1)
  pos_emb = jnp.concatenate([p_o[:BATCH], p_o[BATCH:]], axis=1)
  neg_emb = jnp.concatenate([n_o[:BATCH], n_o[BATCH:]], axis=1)
  return (u_emb, pos_emb, neg_emb)


# speedup vs baseline: 14.9136x; 1.0311x over previous
import functools

import jax
import jax.numpy as jnp
from jax import lax
from jax.experimental import pallas as pl
from jax.experimental.pallas import tpu as pltpu
from jax.experimental.pallas import tpu_sc as plsc

NC = 2
NS = 16
L = 16

N_NODES = 100000
NPAD = 100096
EMB = 32
HALF = EMB // 2
NNZ = 1600000
N_LAYERS = 3
BATCH = 4096

CHUNK = 128
KB = 8
NE_PAD = 1605632
BLOCKS = NE_PAD // (NS * KB * CHUNK)
EROWS = NE_PAD // CHUNK

ROWS_PER_SUB = NPAD // NS
ZCH = 184
ZIT = ROWS_PER_SUB // ZCH

BCH = BATCH // CHUNK
BROWS_PER_SUB = BCH // NS


def _sc_body(e0, col2, row2, val2, users2, pos2, neg2,
             e1, e2, e3, u_out, p_out, n_out,
             acc, col_v, row_v, val_v, cidx_v, fidx_v,
             gbuf_v, g2_v, obuf_v, zbuf_v, sem, ssem):
  c = lax.axis_index("c")
  s = lax.axis_index("s")
  c_off = c * NPAD
  tabs = [e0, e1, e2, e3]

  @pl.loop(0, ZCH)
  def _zero_stage(r):
    zbuf_v[r, :] = jnp.zeros((L,), jnp.float32)

  base_s = s * ROWS_PER_SUB

  for l in range(N_LAYERS):
    src = tabs[l]
    dst = tabs[l + 1]

    @pl.loop(0, ZIT)
    def _clear(z):
      pltpu.sync_copy(zbuf_v, acc.at[pl.ds(base_s + z * ZCH, ZCH), :])

    plsc.subcore_barrier()

    tile_row0 = s * (BLOCKS * KB)

    @pl.loop(0, BLOCKS)
    def _edge_block(b):
      rbase = tile_row0 + b * KB
      pltpu.sync_copy(col2.at[pl.ds(rbase, KB), :], col_v)
      pltpu.sync_copy(row2.at[pl.ds(rbase, KB), :], row_v)
      pltpu.sync_copy(val2.at[pl.ds(rbase, KB), :], val_v)
      for j in range(KB):
        for v in range(CHUNK // L):
          cidx_v[j, pl.ds(v * L, L)] = col_v[j, pl.ds(v * L, L)] + c_off
      gds = [pltpu.async_copy(src.at[cidx_v.at[j]], gbuf_v.at[j], sem)
             for j in range(KB)]
      sds = []
      for j in range(KB):
        gds[j].wait()

        @plsc.parallel_loop(0, CHUNK // L, unroll=2)
        def _scale(g):
          vv = val_v[j, pl.ds(g * L, L)]
          for ee in range(L):
            e = g * L + ee
            gbuf_v[j, e, :] = gbuf_v[j, e, :] * vv[ee]

        sds.append(pltpu.async_copy(gbuf_v.at[j], acc.at[row_v.at[j]],
                                    ssem, add=True))
      for d in sds:
        d.wait()

    plsc.subcore_barrier()

    @pl.loop(0, ZIT)
    def _copy_out(z):
      off = base_s + z * ZCH
      pltpu.sync_copy(acc.at[pl.ds(off, ZCH), :],
                      dst.at[pl.ds(c_off + off, ZCH), :])

    plsc.subcore_barrier()

  for idx_hbm, out_hbm in ((users2, u_out), (pos2, p_out), (neg2, n_out)):
    for rr in range(BROWS_PER_SUB):
      r = s * BROWS_PER_SUB + rr
      pltpu.sync_copy(idx_hbm.at[pl.ds(r * CHUNK, CHUNK)], fidx_v)
      for v in range(CHUNK // L):
        cidx_v[0, pl.ds(v * L, L)] = fidx_v[pl.ds(v * L, L)] + c_off
      pltpu.async_copy(tabs[0].at[cidx_v.at[0]], obuf_v, sem).wait()
      for l in range(1, N_LAYERS + 1):
        pltpu.async_copy(tabs[l].at[cidx_v.at[0]], g2_v, sem).wait()

        @pl.loop(0, CHUNK, unroll=8)
        def _acc_l(e):
          obuf_v[e, :] = obuf_v[e, :] + g2_v[e, :]

      @pl.loop(0, CHUNK, unroll=8)
      def _mean(e):
        obuf_v[e, :] = obuf_v[e, :] * jnp.float32(1.0 / (N_LAYERS + 1))

      pltpu.sync_copy(obuf_v,
                      out_hbm.at[pl.ds(c * BATCH + r * CHUNK, CHUNK), :])


@jax.jit
def _lightgcn(e0, col2, row2, val2, users2, pos2, neg2):
  mesh = plsc.VectorSubcoreMesh(core_axis_name="c", subcore_axis_name="s",
                                num_cores=NC, num_subcores=NS)
  f32 = jnp.float32
  out_type = [
      jax.ShapeDtypeStruct((NC * NPAD, HALF), f32),
      jax.ShapeDtypeStruct((NC * NPAD, HALF), f32),
      jax.ShapeDtypeStruct((NC * NPAD, HALF), f32),
      jax.ShapeDtypeStruct((NC * BATCH, HALF), f32),
      jax.ShapeDtypeStruct((NC * BATCH, HALF), f32),
      jax.ShapeDtypeStruct((NC * BATCH, HALF), f32),
  ]
  scratch = [
      pltpu.VMEM_SHARED((NPAD, HALF), f32),
      pltpu.VMEM((KB, CHUNK), jnp.int32),
      pltpu.VMEM((KB, CHUNK), jnp.int32),
      pltpu.VMEM((KB, CHUNK), f32),
      pltpu.VMEM((KB, CHUNK), jnp.int32),
      pltpu.VMEM((CHUNK,), jnp.int32),
      pltpu.VMEM((KB, CHUNK, HALF), f32),
      pltpu.VMEM((CHUNK, HALF), f32),
      pltpu.VMEM((CHUNK, HALF), f32),
      pltpu.VMEM((ZCH, HALF), f32),
      pltpu.SemaphoreType.DMA,
      pltpu.SemaphoreType.DMA,
  ]
  fn = pl.kernel(_sc_body, out_type=out_type, mesh=mesh,
                 scratch_types=scratch,
                 compiler_params=pltpu.CompilerParams(
                     use_tc_tiling_on_sc=False))
  return fn(e0, col2, row2, val2, users2, pos2, neg2)


def kernel(user_emb, item_emb, adj_val, adj_row, adj_col,
           users, pos_items, neg_items):
  n_user = user_emb.shape[0]
  E = jnp.concatenate([user_emb, item_emb], axis=0)
  zpad = jnp.zeros((NPAD - N_NODES, HALF), jnp.float32)
  e0 = jnp.concatenate([E[:, :HALF], zpad, E[:, HALF:], zpad], axis=0)

  pad = NE_PAD - NNZ
  col2 = jnp.concatenate([adj_col, jnp.zeros((pad,), jnp.int32)])
  row2 = jnp.concatenate([adj_row, jnp.zeros((pad,), jnp.int32)])
  val2 = jnp.concatenate([adj_val, jnp.zeros((pad,), jnp.float32)])
  col2 = col2.reshape(EROWS, CHUNK)
  row2 = row2.reshape(EROWS, CHUNK)
  val2 = val2.reshape(EROWS, CHUNK)

  users2 = users
  pos2 = pos_items + n_user
  neg2 = neg_items + n_user

  _, _, _, u_o, p_o, n_o = _lightgcn(e0, col2, row2, val2,
                                     users2, pos2, neg2)
  u_emb = jnp.concatenate([u_o[:BATCH], u_o[BATCH:]], axis=1)
  pos_emb = jnp.concatenate([p_o[:BATCH], p_o[BATCH:]], axis=1)
  neg_emb = jnp.concatenate([n_o[:BATCH], n_o[BATCH:]], axis=1)
  return (u_emb, pos_emb, neg_emb)
